# baseline (speedup 1.0000x reference)
import functools

import jax
import jax.numpy as jnp
from jax import lax
from jax.experimental import pallas as pl
from jax.experimental.pallas import tpu as pltpu
from jax.experimental.pallas import tpu_sc as plsc

_E_DIM = 32
_BETA = 0.25

_T_BLK = 1024
_C_BLK = 1024
_I32_MAX = 2147483647


_R_SUB = 128
_LANES = 128


def _dist_body(zf, z2, qc, c2, minv, mini, mv_s, mk_s):
    c = pl.program_id(1)
    m = lax.dot_general(zf[...], qc[...], (((1,), (1,)), ((), ())),
                        preferred_element_type=jnp.float32)
    n_k = _C_BLK // _LANES

    @pl.when(c == 0)
    def _():
        mv_s[...] = jnp.full((_T_BLK, _LANES), jnp.inf, jnp.float32)
        mk_s[...] = jnp.zeros((_T_BLK, _LANES), jnp.int32)

    rv = mv_s[...]
    rk = mk_s[...]
    for k in range(n_k):
        k0 = k * _LANES
        u = (z2[...] + c2[:, k0:k0 + _LANES]) - m[:, k0:k0 + _LANES]
        better = u < rv
        rv = jnp.where(better, u, rv)
        rk = jnp.where(better, c * n_k + k, rk)
    mv_s[...] = rv
    mk_s[...] = rk

    @pl.when(c == pl.num_programs(1) - 1)
    def _():
        bmin = jnp.min(rv, axis=1, keepdims=True)
        comp = rk * _LANES + lax.broadcasted_iota(jnp.int32, rk.shape, 1)
        bidx = jnp.min(jnp.where(rv <= bmin, comp, _I32_MAX),
                       axis=1, keepdims=True)
        minv[...] = bmin
        mini[...] = bidx


def _dist_argmin(z_flat, z2, qc, c2r):
    tokens = z_flat.shape[0]
    n_e = qc.shape[0]
    grid = (tokens // _T_BLK, n_e // _C_BLK)
    return pl.pallas_call(
        _dist_body,
        grid=grid,
        in_specs=[
            pl.BlockSpec((_T_BLK, _E_DIM), lambda t, c: (t, 0)),
            pl.BlockSpec((_T_BLK, 1), lambda t, c: (t, 0)),
            pl.BlockSpec((_C_BLK, _E_DIM), lambda t, c: (c, 0)),
            pl.BlockSpec((1, _C_BLK), lambda t, c: (0, c)),
        ],
        out_specs=[
            pl.BlockSpec((_T_BLK, 1), lambda t, c: (t, 0)),
            pl.BlockSpec((_T_BLK, 1), lambda t, c: (t, 0)),
        ],
        out_shape=[
            jax.ShapeDtypeStruct((tokens, 1), jnp.float32),
            jax.ShapeDtypeStruct((tokens, 1), jnp.int32),
        ],
        scratch_shapes=[
            pltpu.VMEM((_T_BLK, _LANES), jnp.float32),
            pltpu.VMEM((_T_BLK, _LANES), jnp.int32),
        ],
        compiler_params=pltpu.CompilerParams(
            dimension_semantics=("parallel", "arbitrary")),
    )(z_flat, z2, qc, c2r)


def _sc_gather(table, idx):
    info = plsc.get_sparse_core_info()
    nc, ns = info.num_cores, info.num_subcores
    nw = nc * ns
    b = idx.shape[0]
    d = table.shape[1]
    ch = 128
    per_w = b // nw
    k = per_w // ch
    idx3 = idx.reshape(nw, k, ch)
    mesh = plsc.VectorSubcoreMesh(core_axis_name="c", subcore_axis_name="s")

    @functools.partial(
        pl.kernel, mesh=mesh,
        out_type=jax.ShapeDtypeStruct((b, d), jnp.float32),
        scratch_types=[
            pltpu.VMEM((k, ch), jnp.int32),
            pltpu.VMEM((per_w, d), jnp.float32),
            pltpu.SemaphoreType.DMA,
        ],
    )
    def g(table_hbm, idx_hbm, out_hbm, idx_v, rows_v, sem):
        wid = lax.axis_index("s") * nc + lax.axis_index("c")
        pltpu.sync_copy(idx_hbm.at[wid], idx_v)
        cps = [pltpu.async_copy(table_hbm.at[idx_v.at[j]],
                                rows_v.at[pl.ds(j * ch, ch)], sem)
               for j in range(k)]
        for cp in cps:
            cp.wait()
        pltpu.sync_copy(rows_v, out_hbm.at[pl.ds(wid * per_w, per_w)])

    return g(table, idx3)


def kernel(z, emb_weight, proj_W, proj_b):
    zc = jnp.transpose(z, (0, 2, 3, 1))
    z_flat = zc.reshape(-1, _E_DIM)
    quant_codebook = emb_weight @ proj_W.T + proj_b
    z2 = jnp.sum(z_flat ** 2, axis=1, keepdims=True)
    c2 = jnp.sum(quant_codebook ** 2, axis=1)

    minv, mini = _dist_argmin(z_flat, 0.5 * z2, quant_codebook,
                              0.5 * c2.reshape(1, -1))
    loss_sum = jnp.sum(minv)
    idx = mini.reshape(-1)
    qc_pad = jnp.pad(quant_codebook, ((0, 0), (0, 128 - _E_DIM)))
    z_q_flat = _sc_gather(qc_pad, idx)[:, :_E_DIM]

    b, h, w, cdim = zc.shape
    z_q_out = jnp.transpose(z_q_flat.reshape(b, h, w, cdim), (0, 3, 1, 2))
    idx_out = idx.reshape(b, h, w)
    n_elems = b * h * w * cdim
    commit_loss = (_BETA + 1.0) * (2.0 * loss_sum) / n_elems
    zero = jnp.zeros((), dtype=jnp.float32)
    return ((z_q_out, zero, idx_out), (zero, zero, commit_loss, zero))


# speedup vs baseline: 1.4158x; 1.4158x over previous
import functools

import jax
import jax.numpy as jnp
from jax import lax
from jax.experimental import pallas as pl
from jax.experimental.pallas import tpu as pltpu
from jax.experimental.pallas import tpu_sc as plsc

_E_DIM = 32
_BETA = 0.25

_T_BLK = 1024
_C_BLK = 1024
_I32_MAX = 2147483647


_R_SUB = 128
_LANES = 128


def _dist_body(zf, z2, qc, c2, minv, mini):
    rv = jnp.full((_T_BLK, _LANES), jnp.inf, jnp.float32)
    rk = jnp.zeros((_T_BLK, _LANES), jnp.int32)
    z2v = z2[...]
    n_k = _C_BLK // _LANES
    for s in range(qc.shape[0] // _C_BLK):
        s0 = s * _C_BLK
        m = lax.dot_general(zf[...], qc[s0:s0 + _C_BLK, :],
                            (((1,), (1,)), ((), ())),
                            preferred_element_type=jnp.float32)
        for k in range(n_k):
            k0 = k * _LANES
            u = (z2v + c2[:, s0 + k0:s0 + k0 + _LANES]) - m[:, k0:k0 + _LANES]
            better = u < rv
            rv = jnp.where(better, u, rv)
            rk = jnp.where(better, s * n_k + k, rk)
    bmin = jnp.min(rv, axis=1, keepdims=True)
    comp = rk * _LANES + lax.broadcasted_iota(jnp.int32, rk.shape, 1)
    bidx = jnp.min(jnp.where(rv <= bmin, comp, _I32_MAX),
                   axis=1, keepdims=True)
    minv[...] = bmin
    mini[...] = bidx


def _dist_argmin(z_flat, z2, qc, c2r):
    tokens = z_flat.shape[0]
    n_e = qc.shape[0]
    grid = (tokens // _T_BLK,)
    return pl.pallas_call(
        _dist_body,
        grid=grid,
        in_specs=[
            pl.BlockSpec((_T_BLK, _E_DIM), lambda t: (t, 0)),
            pl.BlockSpec((_T_BLK, 1), lambda t: (t, 0)),
            pl.BlockSpec((n_e, _E_DIM), lambda t: (0, 0)),
            pl.BlockSpec((1, n_e), lambda t: (0, 0)),
        ],
        out_specs=[
            pl.BlockSpec((_T_BLK, 1), lambda t: (t, 0)),
            pl.BlockSpec((_T_BLK, 1), lambda t: (t, 0)),
        ],
        out_shape=[
            jax.ShapeDtypeStruct((tokens, 1), jnp.float32),
            jax.ShapeDtypeStruct((tokens, 1), jnp.int32),
        ],
        compiler_params=pltpu.CompilerParams(
            dimension_semantics=("parallel",)),
    )(z_flat, z2, qc, c2r)


def _sc_gather(table, idx):
    info = plsc.get_sparse_core_info()
    nc, ns = info.num_cores, info.num_subcores
    nw = nc * ns
    b = idx.shape[0]
    d = table.shape[1]
    ch = 128
    per_w = b // nw
    k = per_w // ch
    idx3 = idx.reshape(nw, k, ch)
    mesh = plsc.VectorSubcoreMesh(core_axis_name="c", subcore_axis_name="s")

    @functools.partial(
        pl.kernel, mesh=mesh,
        out_type=jax.ShapeDtypeStruct((b, d), jnp.float32),
        scratch_types=[
            pltpu.VMEM((k, ch), jnp.int32),
            pltpu.VMEM((per_w, d), jnp.float32),
            pltpu.SemaphoreType.DMA,
        ],
    )
    def g(table_hbm, idx_hbm, out_hbm, idx_v, rows_v, sem):
        wid = lax.axis_index("s") * nc + lax.axis_index("c")
        pltpu.sync_copy(idx_hbm.at[wid], idx_v)
        cps = [pltpu.async_copy(table_hbm.at[idx_v.at[j]],
                                rows_v.at[pl.ds(j * ch, ch)], sem)
               for j in range(k)]
        for cp in cps:
            cp.wait()
        pltpu.sync_copy(rows_v, out_hbm.at[pl.ds(wid * per_w, per_w)])

    return g(table, idx3)


def kernel(z, emb_weight, proj_W, proj_b):
    zc = jnp.transpose(z, (0, 2, 3, 1))
    z_flat = zc.reshape(-1, _E_DIM)
    quant_codebook = emb_weight @ proj_W.T + proj_b
    z2 = jnp.sum(z_flat ** 2, axis=1, keepdims=True)
    c2 = jnp.sum(quant_codebook ** 2, axis=1)

    minv, mini = _dist_argmin(z_flat, 0.5 * z2, quant_codebook,
                              0.5 * c2.reshape(1, -1))
    loss_sum = jnp.sum(minv)
    idx = mini.reshape(-1)
    qc_pad = jnp.pad(quant_codebook, ((0, 0), (0, 128 - _E_DIM)))
    z_q_flat = _sc_gather(qc_pad, idx)[:, :_E_DIM]

    b, h, w, cdim = zc.shape
    z_q_out = jnp.transpose(z_q_flat.reshape(b, h, w, cdim), (0, 3, 1, 2))
    idx_out = idx.reshape(b, h, w)
    n_elems = b * h * w * cdim
    commit_loss = (_BETA + 1.0) * (2.0 * loss_sum) / n_elems
    zero = jnp.zeros((), dtype=jnp.float32)
    return ((z_q_out, zero, idx_out), (zero, zero, commit_loss, zero))
